# baseline (speedup 1.0000x reference)
import functools

import jax
import jax.numpy as jnp
from jax import lax
from jax.experimental import pallas as pl
from jax.experimental.pallas import tpu as pltpu
from jax.experimental.pallas import tpu_sc as plsc

VOCAB = 1000000
EMBED = 64
B = 4096
L = 200
EPS = 1e-5

NC = 2
NS = 16
LANES = 16
NW = NC * NS
N = B * L
PW = N // NW
CH = 128
NCHUNK = PW // CH
NVR = EMBED // LANES


def _rsqrt(x):
    i = plsc.bitcast(x, jnp.int32)
    i = jnp.int32(0x5F3759DF) - lax.shift_right_logical(i, 1)
    y = plsc.bitcast(i, jnp.float32)
    hx = x * jnp.float32(-0.5)
    for _ in range(3):
        y = y * (jnp.float32(1.5) + hx * y * y)
    return y


def _sc_body(ids_hbm, table_hbm, gamma_hbm, beta_hbm, out_hbm,
             idx_v, buf, gb_v, gsem):
    wid = lax.axis_index("s") * NC + lax.axis_index("c")

    pltpu.sync_copy(ids_hbm.at[wid], idx_v)
    pltpu.sync_copy(gamma_hbm, gb_v.at[0])
    pltpu.sync_copy(beta_hbm, gb_v.at[1])

    @pl.loop(0, NCHUNK)
    def _chunk(c):
        pltpu.async_copy(table_hbm.at[idx_v.at[c]], buf, gsem).wait()

        @pl.loop(0, CH)
        def _row(r):
            v = [buf[r, pl.ds(16 * k, 16)] for k in range(NVR)]
            s = (v[0] + v[1]) + (v[2] + v[3])
            sq = (v[0] * v[0] + v[1] * v[1]) + (v[2] * v[2] + v[3] * v[3])
            tot = jnp.sum(s)
            totsq = jnp.sum(sq)
            mean = tot * jnp.float32(1.0 / EMBED)
            var = totsq * jnp.float32(1.0 / EMBED) - mean * mean
            rstd = _rsqrt(jnp.full((16,), var + jnp.float32(EPS),
                                   dtype=jnp.float32))
            for k in range(NVR):
                g = gb_v[0, pl.ds(16 * k, 16)]
                bt = gb_v[1, pl.ds(16 * k, 16)]
                buf[r, pl.ds(16 * k, 16)] = ((v[k] - mean) * rstd) * g + bt

        pltpu.sync_copy(buf, out_hbm.at[pl.ds(wid * PW + c * CH, CH)])


@jax.jit
def _run(ids, table, gamma, beta):
    mesh = plsc.VectorSubcoreMesh(
        core_axis_name="c", subcore_axis_name="s",
        num_cores=NC, num_subcores=NS)
    f = pl.kernel(
        _sc_body,
        out_type=jax.ShapeDtypeStruct((N, EMBED), jnp.float32),
        mesh=mesh,
        compiler_params=pltpu.CompilerParams(
            needs_layout_passes=False, use_tc_tiling_on_sc=False),
        scratch_types=[
            pltpu.VMEM((NCHUNK, CH), jnp.int32),
            pltpu.VMEM((CH, EMBED), jnp.float32),
            pltpu.VMEM((2, EMBED), jnp.float32),
            pltpu.SemaphoreType.DMA,
        ],
    )
    return f(ids, table, gamma, beta)


def kernel(input_ids, table, gamma, beta):
    ids = input_ids.astype(jnp.int32).reshape(NW, NCHUNK, CH)
    out = _run(ids, table, gamma, beta)
    return out.reshape(B, L, EMBED)


# speedup vs baseline: 1.5991x; 1.5991x over previous
import jax
import jax.numpy as jnp
from jax import lax
from jax.experimental import pallas as pl
from jax.experimental.pallas import tpu as pltpu
from jax.experimental.pallas import tpu_sc as plsc

VOCAB = 1000000
EMBED = 64
B = 4096
L = 200
EPS = 1e-5

NC = 2
NS = 16
NW = NC * NS
N = B * L
PW = N // NW
CH = 128
NCHUNK = PW // CH
NVR = EMBED // 16
NBUF = 4


def _rsqrt(x):
    i = lax.bitcast_convert_type(x, jnp.int32)
    i = jnp.int32(0x5F3759DF) - lax.shift_right_logical(i, 1)
    y = lax.bitcast_convert_type(i, jnp.float32)
    hx = x * jnp.float32(-0.5)
    for _ in range(2):
        y = y * (jnp.float32(1.5) + hx * y * y)
    return y


def _sc_body(ids_hbm, table_hbm, gamma_hbm, beta_hbm, out_hbm,
             idx_v, bufs, gb_v, gsem, osem):
    wid = lax.axis_index("s") * NC + lax.axis_index("c")
    base = wid * PW

    pltpu.sync_copy(ids_hbm.at[wid], idx_v)
    pltpu.sync_copy(gamma_hbm, gb_v.at[0])
    pltpu.sync_copy(beta_hbm, gb_v.at[1])

    def start_gather(g, b):
        pltpu.async_copy(table_hbm.at[idx_v.at[g]], bufs.at[b], gsem.at[b])

    def wait_gather(g, b):
        pltpu.make_async_copy(
            table_hbm.at[idx_v.at[g]], bufs.at[b], gsem.at[b]).wait()

    def start_out(g, b):
        pltpu.async_copy(bufs.at[b], out_hbm.at[pl.ds(base + g * CH, CH)],
                         osem.at[b])

    def wait_out(g, b):
        pltpu.make_async_copy(
            bufs.at[b], out_hbm.at[pl.ds(base + g * CH, CH)],
            osem.at[b]).wait()

    def layernorm_chunk(b):
        buf = bufs.at[b]

        @pl.loop(0, CH, unroll=2)
        def _row(r):
            v = [buf[r, pl.ds(16 * k, 16)] for k in range(NVR)]
            s = (v[0] + v[1]) + (v[2] + v[3])
            q = (v[0] * v[0] + v[1] * v[1]) + (v[2] * v[2] + v[3] * v[3])
            mean = jnp.sum(s) * jnp.float32(1.0 / EMBED)
            var = jnp.sum(q) * jnp.float32(1.0 / EMBED) - mean * mean
            rstd = _rsqrt(var + jnp.float32(EPS))
            for k in range(NVR):
                g = gb_v[0, pl.ds(16 * k, 16)]
                bt = gb_v[1, pl.ds(16 * k, 16)]
                buf[r, pl.ds(16 * k, 16)] = ((v[k] - mean) * rstd) * g + bt

    for g in range(2):
        start_gather(g, g)

    for g in range(2):
        wait_gather(g, g)
        layernorm_chunk(g)
        start_out(g, g)
        start_gather(g + 2, (g + 2) % NBUF)

    @pl.loop(0, (NCHUNK - 4) // NBUF)
    def _outer(go):
        for b in range(NBUF):
            g = 2 + go * NBUF + b
            bufi = (2 + b) % NBUF
            wait_out(g - 2, b)
            start_gather(g + 2, b)
            wait_gather(g, bufi)
            layernorm_chunk(bufi)
            start_out(g, bufi)

    for g in range(NCHUNK - 2, NCHUNK):
        bufi = g % NBUF
        wait_gather(g, bufi)
        layernorm_chunk(bufi)
        start_out(g, bufi)
    for g in range(NCHUNK - 4, NCHUNK):
        wait_out(g, g % NBUF)


@jax.jit
def _run(ids, table, gamma, beta):
    mesh = plsc.VectorSubcoreMesh(
        core_axis_name="c", subcore_axis_name="s",
        num_cores=NC, num_subcores=NS)
    f = pl.kernel(
        _sc_body,
        out_type=jax.ShapeDtypeStruct((N, EMBED), jnp.float32),
        mesh=mesh,
        compiler_params=pltpu.CompilerParams(
            needs_layout_passes=False, use_tc_tiling_on_sc=False),
        scratch_types=[
            pltpu.VMEM((NCHUNK, CH), jnp.int32),
            pltpu.VMEM((NBUF, CH, EMBED), jnp.float32),
            pltpu.VMEM((2, EMBED), jnp.float32),
            pltpu.SemaphoreType.DMA((NBUF,)),
            pltpu.SemaphoreType.DMA((NBUF,)),
        ],
    )
    return f(ids, table, gamma, beta)


def kernel(input_ids, table, gamma, beta):
    ids = input_ids.astype(jnp.int32).reshape(NW, NCHUNK, CH)
    out = _run(ids, table, gamma, beta)
    return out.reshape(B, L, EMBED)
